# baseline (speedup 1.0000x reference)
import jax
import jax.numpy as jnp
from jax.experimental import pallas as pl
from jax.experimental.pallas import tpu as pltpu

S = 2048
D = 1024
NC_PER_B = 8
CR = S // NC_PER_B
NC = 2 * NC_PER_B


def _body(bos_ref, idx_ref, embeds_ref, speech_ref, pos_ref, out_ref,
          buf, row_a, row_b, rsem, wsem, sem_a, sem_b):
    reads = []
    for k in range(NC):
        b, c = divmod(k, NC_PER_B)
        cp = pltpu.make_async_copy(
            embeds_ref.at[b, pl.ds(c * CR, CR), :],
            buf.at[k], rsem.at[k])
        cp.start()
        reads.append(cp)

    tok = bos_ref[0, 0]
    ix = idx_ref[0]
    cp_a = pltpu.make_async_copy(speech_ref.at[pl.ds(tok, 1), :], row_a, sem_a)
    cp_b = pltpu.make_async_copy(pos_ref.at[pl.ds(ix, 1), :], row_b, sem_b)
    cp_a.start()
    cp_b.start()

    writes = []
    for k in range(NC):
        b, c = divmod(k, NC_PER_B)
        reads[k].wait()
        cp = pltpu.make_async_copy(
            buf.at[k],
            out_ref.at[b, pl.ds(c * CR, CR), :], wsem.at[k])
        cp.start()
        writes.append(cp)

    cp_a.wait()
    cp_b.wait()
    row_a[...] = row_a[...] + row_b[...]
    cp0 = pltpu.make_async_copy(row_a, out_ref.at[0, pl.ds(S, 1), :], sem_a)
    cp1 = pltpu.make_async_copy(row_a, out_ref.at[1, pl.ds(S, 1), :], sem_b)
    cp0.start()
    cp1.start()
    cp0.wait()
    cp1.wait()
    for cp in writes:
        cp.wait()


def kernel(bos_token, embeds, idx, speech_emb, pos_emb):
    out = pl.pallas_call(
        _body,
        out_shape=jax.ShapeDtypeStruct((2, S + 1, D), jnp.float32),
        in_specs=[
            pl.BlockSpec(memory_space=pltpu.SMEM),
            pl.BlockSpec(memory_space=pltpu.SMEM),
            pl.BlockSpec(memory_space=pl.ANY),
            pl.BlockSpec(memory_space=pl.ANY),
            pl.BlockSpec(memory_space=pl.ANY),
        ],
        out_specs=pl.BlockSpec(memory_space=pl.ANY),
        scratch_shapes=[
            pltpu.VMEM((NC, CR, D), jnp.float32),
            pltpu.VMEM((1, D), jnp.float32),
            pltpu.VMEM((1, D), jnp.float32),
            pltpu.SemaphoreType.DMA((NC,)),
            pltpu.SemaphoreType.DMA((NC,)),
            pltpu.SemaphoreType.DMA,
            pltpu.SemaphoreType.DMA,
        ],
    )(bos_token, idx, embeds, speech_emb, pos_emb)
    return out
